# Initial kernel scaffold; baseline (speedup 1.0000x reference)
import jax
import jax.numpy as jnp
from jax.experimental import pallas as pl


def kernel(betas, alphas, alpha_bars, num_steps):
    raise NotImplementedError("write your pallas kernel here")


# speedup vs baseline: 12.3040x; 12.3040x over previous
import functools

import jax
import jax.numpy as jnp
from jax import lax
from jax.experimental import pallas as pl
from jax.experimental.pallas import tpu as pltpu
from jax.experimental.pallas import tpu_sc as plsc

_MAX_STEPS = 1000
_TAB = 1024
_B = 16384
_NC = 2
_NS = 16
_L = 16
_NW = _NC * _NS
_BPW = _B // _NW

_mesh = plsc.VectorSubcoreMesh(core_axis_name="c", subcore_axis_name="s")


@functools.partial(
    pl.kernel,
    mesh=_mesh,
    compiler_params=pltpu.CompilerParams(needs_layout_passes=False),
    out_type=jax.ShapeDtypeStruct((3 * _B,), jnp.float32),
    scratch_types=[
        pltpu.VMEM((3 * _TAB,), jnp.float32),
        pltpu.VMEM((_BPW,), jnp.int32),
        pltpu.VMEM((3 * _BPW,), jnp.float32),
    ],
)
def _lookup(tables_hbm, idx_hbm, out_hbm, tab_v, idx_v, out_v):
    wid = lax.axis_index("s") * _NC + lax.axis_index("c")
    base = wid * _BPW

    pltpu.sync_copy(tables_hbm, tab_v)
    pltpu.sync_copy(idx_hbm.at[pl.ds(base, _BPW)], idx_v)

    def body(i, carry):
        sl = pl.ds(i * _L, _L)
        idx = idx_v[sl]
        out_v[sl] = plsc.load_gather(tab_v, [idx])
        out_v[pl.ds(_BPW + i * _L, _L)] = plsc.load_gather(tab_v, [idx + _TAB])
        out_v[pl.ds(2 * _BPW + i * _L, _L)] = plsc.load_gather(
            tab_v, [idx + 2 * _TAB]
        )
        return carry

    lax.fori_loop(0, _BPW // _L, body, 0)

    for c in range(3):
        pltpu.sync_copy(
            out_v.at[pl.ds(c * _BPW, _BPW)],
            out_hbm.at[pl.ds(c * _B + base, _BPW)],
        )


def kernel(betas, alphas, alpha_bars, num_steps):
    tables = jnp.pad(
        jnp.stack([betas, alphas, alpha_bars], axis=0),
        ((0, 0), (0, _TAB - _MAX_STEPS)),
    ).reshape(-1)
    flat = _lookup(tables, num_steps.astype(jnp.int32))
    return flat.reshape(3, _B)


# speedup vs baseline: 12.3420x; 1.0031x over previous
import functools

import jax
import jax.numpy as jnp
from jax import lax
from jax.experimental import pallas as pl
from jax.experimental.pallas import tpu as pltpu
from jax.experimental.pallas import tpu_sc as plsc

_MAX_STEPS = 1000
_TAB = 1024
_B = 16384
_NC = 2
_NS = 16
_L = 16
_NW = _NC * _NS
_BPW = _B // _NW

_mesh = plsc.VectorSubcoreMesh(core_axis_name="c", subcore_axis_name="s")


@functools.partial(
    pl.kernel,
    mesh=_mesh,
    compiler_params=pltpu.CompilerParams(needs_layout_passes=False),
    out_type=jax.ShapeDtypeStruct((3 * _B,), jnp.float32),
    scratch_types=[
        pltpu.VMEM((3 * _TAB,), jnp.float32),
        pltpu.VMEM((_BPW,), jnp.int32),
        pltpu.VMEM((3 * _BPW,), jnp.float32),
        pltpu.SemaphoreType.DMA,
    ],
)
def _lookup(tables_hbm, idx_hbm, out_hbm, tab_v, idx_v, out_v, sem):
    wid = lax.axis_index("s") * _NC + lax.axis_index("c")
    base = wid * _BPW

    cp_tab = pltpu.async_copy(tables_hbm, tab_v, sem)
    cp_idx = pltpu.async_copy(idx_hbm.at[pl.ds(base, _BPW)], idx_v, sem)
    cp_tab.wait()
    cp_idx.wait()

    for i in range(_BPW // _L):
        sl = pl.ds(i * _L, _L)
        idx = idx_v[sl]
        out_v[sl] = plsc.load_gather(tab_v, [idx])
        out_v[pl.ds(_BPW + i * _L, _L)] = plsc.load_gather(tab_v, [idx + _TAB])
        out_v[pl.ds(2 * _BPW + i * _L, _L)] = plsc.load_gather(
            tab_v, [idx + 2 * _TAB]
        )

    cps = [
        pltpu.async_copy(
            out_v.at[pl.ds(c * _BPW, _BPW)],
            out_hbm.at[pl.ds(c * _B + base, _BPW)],
            sem,
        )
        for c in range(3)
    ]
    for cp in cps:
        cp.wait()


def kernel(betas, alphas, alpha_bars, num_steps):
    tables = jnp.pad(
        jnp.stack([betas, alphas, alpha_bars], axis=0),
        ((0, 0), (0, _TAB - _MAX_STEPS)),
    ).reshape(-1)
    flat = _lookup(tables, num_steps.astype(jnp.int32))
    return flat.reshape(3, _B)
